# Initial kernel scaffold; baseline (speedup 1.0000x reference)
import functools

import jax
import jax.numpy as jnp
from jax.experimental import pallas as pl

_SIZE = 512
_LENGTH = 16
_ATT = 256
_HID = 512
_R = 1024


def _transition_kernel(init_ref, subj_ref, obj_ref, relenc_ref, wrel_ref,
                       brel_ref, mm0_ref, w1_ref, b1_ref, w2_ref, b2_ref,
                       out_ref):
    f32 = jnp.float32
    hp = jax.lax.Precision.HIGHEST
    dot = functools.partial(jnp.dot, precision=hp, preferred_element_type=f32)
    dot_t = functools.partial(jax.lax.dot_general,
                              dimension_numbers=(((1,), (1,)), ((), ())),
                              precision=hp, preferred_element_type=f32)

    row_ids = jax.lax.broadcasted_iota(jnp.int32, (_SIZE, _R), 0)
    gath = (row_ids == subj_ref[:]).astype(f32)
    scat = (row_ids == obj_ref[:]).astype(f32)
    relation = dot(relenc_ref[:], wrel_ref[:]) + brel_ref[:]

    state = init_ref[:]
    bsz = state.shape[0]
    meta = jnp.broadcast_to(mm0_ref[:], (bsz, _ATT))
    w1a = w1_ref[:_SIZE, :]
    w1b = w1_ref[_SIZE:, :]

    for i in range(_LENGTH):
        hid = jnp.maximum(dot(state, w1a) + dot(meta, w1b) + b1_ref[:], 0.0)
        meta = dot(hid, w2_ref[:]) + b2_ref[:]
        h = jax.nn.sigmoid(dot_t(meta, relation))
        gathered = dot(state, gath)
        state = dot_t(gathered * h, scat)
        out_ref[:, i, :] = state


def kernel(x, rel_subj, rel_obj, rel_enc, Wrel, brel, action_table, pos_table,
           metaMode_init, W1G, b1G, W2G, b2G):
    bsz = x.shape[0]
    init = x[:, :_SIZE].astype(jnp.float32)
    return pl.pallas_call(
        _transition_kernel,
        out_shape=jax.ShapeDtypeStruct((bsz, _LENGTH, _SIZE), jnp.float32),
    )(init, rel_subj.reshape(1, _R), rel_obj.reshape(1, _R), rel_enc[:_R],
      Wrel, brel.reshape(1, _ATT), metaMode_init.reshape(1, _ATT), W1G,
      b1G.reshape(1, _HID), W2G, b2G.reshape(1, _ATT))


# speedup vs baseline: 2.7198x; 2.7198x over previous
import jax
import jax.numpy as jnp
from jax.experimental import pallas as pl
from jax.experimental.pallas import tpu as pltpu

_SIZE = 512
_LENGTH = 16
_ATT = 256
_R = 1024


def _gather_scatter_kernel(subj_ref, obj_ref, state_t_ref, h_t_ref, out_ref):
    out_ref[:] = jnp.zeros_like(out_ref)

    def body(j, carry):
        sj = subj_ref[0, j]
        oj = obj_ref[0, j]
        row = state_t_ref[pl.ds(sj, 1), :] * h_t_ref[pl.ds(j, 1), :]
        out_ref[pl.ds(oj, 1), :] = out_ref[pl.ds(oj, 1), :] + row
        return carry

    jax.lax.fori_loop(0, _R, body, 0)


def kernel(x, rel_subj, rel_obj, rel_enc, Wrel, brel, action_table, pos_table,
           metaMode_init, W1G, b1G, W2G, b2G):
    bsz = x.shape[0]
    step = pl.pallas_call(
        _gather_scatter_kernel,
        in_specs=[
            pl.BlockSpec(memory_space=pltpu.SMEM),
            pl.BlockSpec(memory_space=pltpu.SMEM),
            pl.BlockSpec(memory_space=pltpu.VMEM),
            pl.BlockSpec(memory_space=pltpu.VMEM),
        ],
        out_specs=pl.BlockSpec(memory_space=pltpu.VMEM),
        out_shape=jax.ShapeDtypeStruct((_SIZE, bsz), jnp.float32),
    )
    subj = rel_subj.reshape(1, _R)
    obj = rel_obj.reshape(1, _R)

    state = x[:, :_SIZE].astype(jnp.float32)
    metaMode = jnp.broadcast_to(metaMode_init[None], (bsz, _ATT))
    relation = jnp.dot(rel_enc[:_R], Wrel) + brel
    outs = []
    for _ in range(_LENGTH):
        g_in = jnp.concatenate((state, metaMode), axis=1)
        metaMode = jax.nn.relu(jnp.dot(g_in, W1G) + b1G)
        metaMode = jnp.dot(metaMode, W2G) + b2G
        h = jax.nn.sigmoid(jnp.dot(metaMode, relation.T))
        state = step(subj, obj, state.T, h.T).T
        outs.append(state)
    return jnp.stack(outs, axis=1)
